# baseline (speedup 1.0000x reference)
import jax
import jax.numpy as jnp
from jax.experimental import pallas as pl

_NBLK = 10240


def _body(f_ref, d_ref, w1_ref, b1_ref, wc_ref, bc_ref, wr_ref, br_ref,
          sem_ref, ang_ref, dist_ref):
    f = f_ref[0]
    x = jnp.dot(w1_ref[...], f, preferred_element_type=jnp.float32)
    x = jnp.maximum(x + b1_ref[...], 0.0)
    sem_ref[0] = jnp.dot(wc_ref[...], x, preferred_element_type=jnp.float32) + bc_ref[...]
    reg = jnp.dot(wr_ref[...], x, preferred_element_type=jnp.float32) + br_ref[...]
    ang_ref[0] = reg[0:1]
    dist_ref[0] = d_ref[0] + reg[1:7]


def kernel(fused_feats, obj_scores, distance, W1, b1, gamma1, beta1, Wc, bc, Wr, br):
    B, C, N = fused_feats.shape
    NUM_CLS = Wc.shape[0]
    W1f = W1 * gamma1[:, None]
    b1f = (b1 * gamma1 + beta1)[:, None]
    nb = pl.cdiv(N, _NBLK)

    grid = (B, nb)
    out_shapes = (
        jax.ShapeDtypeStruct((B, NUM_CLS, N), jnp.float32),
        jax.ShapeDtypeStruct((B, 1, N), jnp.float32),
        jax.ShapeDtypeStruct((B, 6, N), jnp.float32),
    )
    sem_cm, ang, dist_cm = pl.pallas_call(
        _body,
        grid=grid,
        in_specs=[
            pl.BlockSpec((1, C, _NBLK), lambda b, n: (b, 0, n)),
            pl.BlockSpec((1, 6, _NBLK), lambda b, n: (b, 0, n)),
            pl.BlockSpec((C, C), lambda b, n: (0, 0)),
            pl.BlockSpec((C, 1), lambda b, n: (0, 0)),
            pl.BlockSpec((NUM_CLS, C), lambda b, n: (0, 0)),
            pl.BlockSpec((NUM_CLS, 1), lambda b, n: (0, 0)),
            pl.BlockSpec((7, C), lambda b, n: (0, 0)),
            pl.BlockSpec((7, 1), lambda b, n: (0, 0)),
        ],
        out_specs=(
            pl.BlockSpec((1, NUM_CLS, _NBLK), lambda b, n: (b, 0, n)),
            pl.BlockSpec((1, 1, _NBLK), lambda b, n: (b, 0, n)),
            pl.BlockSpec((1, 6, _NBLK), lambda b, n: (b, 0, n)),
        ),
        out_shape=out_shapes,
    )(fused_feats, jnp.transpose(distance, (0, 2, 1)), W1f, b1f,
      Wc, bc[:, None], Wr, br[:, None])
    sem = jnp.transpose(sem_cm, (0, 2, 1))
    dist = jnp.transpose(dist_cm, (0, 2, 1))
    return (sem, ang.reshape(B, N), dist, obj_scores)


# speedup vs baseline: 1.0301x; 1.0301x over previous
import jax
import jax.numpy as jnp
from jax.experimental import pallas as pl

_NBLK = 20000


def _body(f_ref, d_ref, w1_ref, b1_ref, wc_ref, bc_ref, wr_ref, br_ref,
          sem_ref, ang_ref, dist_ref):
    f = f_ref[0].astype(jnp.bfloat16)
    x = jnp.dot(w1_ref[...].astype(jnp.bfloat16), f,
                preferred_element_type=jnp.float32)
    x = jnp.maximum(x + b1_ref[...], 0.0).astype(jnp.bfloat16)
    sem_ref[0] = jnp.dot(wc_ref[...].astype(jnp.bfloat16), x,
                         preferred_element_type=jnp.float32) + bc_ref[...]
    reg = jnp.dot(wr_ref[...].astype(jnp.bfloat16), x,
                  preferred_element_type=jnp.float32) + br_ref[...]
    ang_ref[0] = reg[0:1]
    dist_ref[0] = d_ref[0] + reg[1:7]


def kernel(fused_feats, obj_scores, distance, W1, b1, gamma1, beta1, Wc, bc, Wr, br):
    B, C, N = fused_feats.shape
    NUM_CLS = Wc.shape[0]
    W1f = W1 * gamma1[:, None]
    b1f = (b1 * gamma1 + beta1)[:, None]
    nb = pl.cdiv(N, _NBLK)

    grid = (B, nb)
    out_shapes = (
        jax.ShapeDtypeStruct((B, NUM_CLS, N), jnp.float32),
        jax.ShapeDtypeStruct((B, 1, N), jnp.float32),
        jax.ShapeDtypeStruct((B, 6, N), jnp.float32),
    )
    sem_cm, ang, dist_cm = pl.pallas_call(
        _body,
        grid=grid,
        in_specs=[
            pl.BlockSpec((1, C, _NBLK), lambda b, n: (b, 0, n)),
            pl.BlockSpec((1, 6, _NBLK), lambda b, n: (b, 0, n)),
            pl.BlockSpec((C, C), lambda b, n: (0, 0)),
            pl.BlockSpec((C, 1), lambda b, n: (0, 0)),
            pl.BlockSpec((NUM_CLS, C), lambda b, n: (0, 0)),
            pl.BlockSpec((NUM_CLS, 1), lambda b, n: (0, 0)),
            pl.BlockSpec((7, C), lambda b, n: (0, 0)),
            pl.BlockSpec((7, 1), lambda b, n: (0, 0)),
        ],
        out_specs=(
            pl.BlockSpec((1, NUM_CLS, _NBLK), lambda b, n: (b, 0, n)),
            pl.BlockSpec((1, 1, _NBLK), lambda b, n: (b, 0, n)),
            pl.BlockSpec((1, 6, _NBLK), lambda b, n: (b, 0, n)),
        ),
        out_shape=out_shapes,
    )(fused_feats, jnp.transpose(distance, (0, 2, 1)), W1f, b1f,
      Wc, bc[:, None], Wr, br[:, None])
    sem = jnp.transpose(sem_cm, (0, 2, 1))
    dist = jnp.transpose(dist_cm, (0, 2, 1))
    return (sem, ang.reshape(B, N), dist, obj_scores)
